# baseline (speedup 1.0000x reference)
import functools

import jax
import jax.numpy as jnp
from jax import lax
from jax.experimental import pallas as pl
from jax.experimental.pallas import tpu as pltpu
from jax.experimental.pallas import tpu_sc as plsc

BATCH = 16384
EMB = 64
HIDDEN = 32

NC = 2
NS = 16
NW = NC * NS
B_PER_W = BATCH // NW


def _sc_gather(E1, E2, idx1, idx2):
    mesh = plsc.VectorSubcoreMesh(core_axis_name="c", subcore_axis_name="s")

    @functools.partial(
        pl.kernel,
        mesh=mesh,
        out_type=(
            jax.ShapeDtypeStruct((BATCH, EMB), jnp.float32),
            jax.ShapeDtypeStruct((BATCH, EMB), jnp.float32),
        ),
        scratch_types=[
            pltpu.VMEM((B_PER_W,), jnp.int32),
            pltpu.VMEM((B_PER_W,), jnp.int32),
            pltpu.VMEM((B_PER_W, EMB), jnp.float32),
            pltpu.VMEM((B_PER_W, EMB), jnp.float32),
            pltpu.SemaphoreType.DMA,
            pltpu.SemaphoreType.DMA,
        ],
        compiler_params=pltpu.CompilerParams(use_tc_tiling_on_sc=False),
    )
    def k(e1_hbm, e2_hbm, i1_hbm, i2_hbm, o1_hbm, o2_hbm,
          i1_v, i2_v, r1_v, r2_v, s1, s2):
        wid = lax.axis_index("s") * NC + lax.axis_index("c")
        base = wid * B_PER_W
        pltpu.sync_copy(i1_hbm.at[pl.ds(base, B_PER_W)], i1_v)
        pltpu.sync_copy(i2_hbm.at[pl.ds(base, B_PER_W)], i2_v)
        c1 = pltpu.async_copy(e1_hbm.at[i1_v], r1_v, s1)
        c2 = pltpu.async_copy(e2_hbm.at[i2_v], r2_v, s2)
        c1.wait()
        c2.wait()
        pltpu.sync_copy(r1_v, o1_hbm.at[pl.ds(base, B_PER_W)])
        pltpu.sync_copy(r2_v, o2_hbm.at[pl.ds(base, B_PER_W)])

    return k(E1, E2, idx1, idx2)


def _mlp_body(x1_ref, x2_ref, w1a_ref, w1b_ref, b1_ref, w2_ref, b2_ref, o_ref):
    h = (
        jnp.dot(x1_ref[...], w1a_ref[...], preferred_element_type=jnp.float32)
        + jnp.dot(x2_ref[...], w1b_ref[...], preferred_element_type=jnp.float32)
        + b1_ref[...]
    )
    h = jnp.maximum(h, 0.0)
    o = jnp.dot(h, w2_ref[...], preferred_element_type=jnp.float32) + b2_ref[...]
    o_ref[...] = jax.nn.sigmoid(o)


def _tc_mlp(x1, x2, W1a, W1b, b1, W2, b2):
    BLK = 4096
    grid = (BATCH // BLK,)
    return pl.pallas_call(
        _mlp_body,
        grid=grid,
        in_specs=[
            pl.BlockSpec((BLK, EMB), lambda i: (i, 0)),
            pl.BlockSpec((BLK, EMB), lambda i: (i, 0)),
            pl.BlockSpec((EMB, HIDDEN), lambda i: (0, 0)),
            pl.BlockSpec((EMB, HIDDEN), lambda i: (0, 0)),
            pl.BlockSpec((1, HIDDEN), lambda i: (0, 0)),
            pl.BlockSpec((HIDDEN, 1), lambda i: (0, 0)),
            pl.BlockSpec((1, 1), lambda i: (0, 0)),
        ],
        out_specs=pl.BlockSpec((BLK, 1), lambda i: (i, 0)),
        out_shape=jax.ShapeDtypeStruct((BATCH, 1), jnp.float32),
    )(x1, x2, W1a, W1b, b1, W2, b2)


def kernel(inputs, E1, E2, W1, b1, W2, b2):
    idx1 = inputs[:, 0]
    idx2 = inputs[:, 1]
    x1, x2 = _sc_gather(E1, E2, idx1, idx2)
    W1a = W1[:EMB]
    W1b = W1[EMB:]
    return _tc_mlp(x1, x2, W1a, W1b, b1.reshape(1, HIDDEN), W2,
                   b2.reshape(1, 1))


# speedup vs baseline: 1.5545x; 1.5545x over previous
import functools

import jax
import jax.numpy as jnp
from jax import lax
from jax.experimental import pallas as pl
from jax.experimental.pallas import tpu as pltpu
from jax.experimental.pallas import tpu_sc as plsc

BATCH = 16384
VOCAB = 1000000
EMB = 64
HIDDEN = 32

BLKV = 8192
NVBLK = (VOCAB + BLKV - 1) // BLKV
VOCABP = NVBLK * BLKV

NC = 2
NS = 16
NW = NC * NS
B_PER_W = BATCH // NW


def _embed_matmul_body(et1_ref, et2_ref, w1a_ref, w1b_ref, h_ref):
    dn = (((0,), (0,)), ((), ()))
    h1 = lax.dot_general(et1_ref[...], w1a_ref[...], dn,
                         preferred_element_type=jnp.float32)
    h2 = lax.dot_general(et2_ref[...], w1b_ref[...], dn,
                         preferred_element_type=jnp.float32)
    h_ref[...] = jnp.concatenate([h1, h2, h1, h2], axis=1)


def _embed_matmul(ET1, ET2, W1a, W1b):
    return pl.pallas_call(
        _embed_matmul_body,
        grid=(NVBLK,),
        in_specs=[
            pl.BlockSpec((EMB, BLKV), lambda i: (0, i)),
            pl.BlockSpec((EMB, BLKV), lambda i: (0, i)),
            pl.BlockSpec((EMB, HIDDEN), lambda i: (0, 0)),
            pl.BlockSpec((EMB, HIDDEN), lambda i: (0, 0)),
        ],
        out_specs=pl.BlockSpec((BLKV, 128), lambda i: (i, 0)),
        out_shape=jax.ShapeDtypeStruct((VOCABP, 128), jnp.float32),
        compiler_params=pltpu.CompilerParams(
            dimension_semantics=("arbitrary",),
        ),
    )(ET1, ET2, W1a, W1b)


def _sc_gather_one(H, idx):
    mesh = plsc.VectorSubcoreMesh(core_axis_name="c", subcore_axis_name="s")

    @functools.partial(
        pl.kernel,
        mesh=mesh,
        out_type=jax.ShapeDtypeStruct((BATCH, 128), jnp.float32),
        scratch_types=[
            pltpu.VMEM((B_PER_W,), jnp.int32),
            pltpu.VMEM((B_PER_W, 128), jnp.float32),
            pltpu.SemaphoreType.DMA,
        ],
        compiler_params=pltpu.CompilerParams(use_tc_tiling_on_sc=False),
    )
    def k(h_hbm, i_hbm, o_hbm, i_v, r_v, s):
        wid = lax.axis_index("s") * NC + lax.axis_index("c")
        base = wid * B_PER_W
        pltpu.sync_copy(i_hbm.at[pl.ds(base, B_PER_W)], i_v)
        pltpu.async_copy(h_hbm.at[i_v], r_v, s).wait()
        pltpu.sync_copy(r_v, o_hbm.at[pl.ds(base, B_PER_W)])

    return k(H, idx)


def _mlp_body(g1_ref, g2_ref, b1_ref, w2_ref, b2_ref, o_ref):
    h = jnp.maximum(
        g1_ref[:, :HIDDEN] + g2_ref[:, HIDDEN:2 * HIDDEN] + b1_ref[...], 0.0)
    o = jnp.dot(h, w2_ref[...], preferred_element_type=jnp.float32) + b2_ref[...]
    o_ref[...] = jax.nn.sigmoid(o)


def _tc_mlp(g1, g2, b1, W2, b2):
    BLK = 4096
    return pl.pallas_call(
        _mlp_body,
        grid=(BATCH // BLK,),
        in_specs=[
            pl.BlockSpec((BLK, 128), lambda i: (i, 0)),
            pl.BlockSpec((BLK, 128), lambda i: (i, 0)),
            pl.BlockSpec((1, HIDDEN), lambda i: (0, 0)),
            pl.BlockSpec((HIDDEN, 1), lambda i: (0, 0)),
            pl.BlockSpec((1, 1), lambda i: (0, 0)),
        ],
        out_specs=pl.BlockSpec((BLK, 1), lambda i: (i, 0)),
        out_shape=jax.ShapeDtypeStruct((BATCH, 1), jnp.float32),
    )(g1, g2, b1, W2, b2)


def kernel(inputs, E1, E2, W1, b1, W2, b2):
    idx1 = inputs[:, 0]
    idx2 = inputs[:, 1]
    ET1 = E1.T
    ET2 = E2.T
    W1a = W1[:EMB]
    W1b = W1[EMB:]
    H = _embed_matmul(ET1, ET2, W1a, W1b)
    g1 = _sc_gather_one(H, idx1)
    g2 = _sc_gather_one(H, idx2)
    return _tc_mlp(g1, g2, b1.reshape(1, HIDDEN), W2, b2.reshape(1, 1))


# speedup vs baseline: 1.5551x; 1.0004x over previous
import functools

import jax
import jax.numpy as jnp
from jax import lax
from jax.experimental import pallas as pl
from jax.experimental.pallas import tpu as pltpu
from jax.experimental.pallas import tpu_sc as plsc

BATCH = 16384
VOCAB = 1000000
EMB = 64
HIDDEN = 32

BLKV = 8192
NVBLK = (VOCAB + BLKV - 1) // BLKV
VOCABP = NVBLK * BLKV

NC = 2
NS = 16
NW = NC * NS
B_PER_W = BATCH // NW


def _embed_matmul_body(et1_ref, et2_ref, w1a_ref, w1b_ref, h_ref):
    dn = (((0,), (0,)), ((), ()))
    h1 = lax.dot_general(et1_ref[...], w1a_ref[...], dn,
                         preferred_element_type=jnp.float32)
    h2 = lax.dot_general(et2_ref[...], w1b_ref[...], dn,
                         preferred_element_type=jnp.float32)
    h_ref[...] = jnp.concatenate([h1, h2, h1, h2], axis=1)


def _embed_matmul(ET1, ET2, W1a, W1b):
    return pl.pallas_call(
        _embed_matmul_body,
        grid=(NVBLK,),
        in_specs=[
            pl.BlockSpec((EMB, BLKV), lambda i: (0, i)),
            pl.BlockSpec((EMB, BLKV), lambda i: (0, i)),
            pl.BlockSpec((EMB, HIDDEN), lambda i: (0, 0)),
            pl.BlockSpec((EMB, HIDDEN), lambda i: (0, 0)),
        ],
        out_specs=pl.BlockSpec((BLKV, 128), lambda i: (i, 0)),
        out_shape=jax.ShapeDtypeStruct((VOCABP, 128), jnp.float32),
        compiler_params=pltpu.CompilerParams(
            dimension_semantics=("parallel",),
        ),
    )(ET1, ET2, W1a, W1b)


def _sc_gather_one(H, idx):
    mesh = plsc.VectorSubcoreMesh(core_axis_name="c", subcore_axis_name="s")

    @functools.partial(
        pl.kernel,
        mesh=mesh,
        out_type=jax.ShapeDtypeStruct((BATCH, 128), jnp.float32),
        scratch_types=[
            pltpu.VMEM((B_PER_W,), jnp.int32),
            pltpu.VMEM((B_PER_W, 128), jnp.float32),
            pltpu.SemaphoreType.DMA,
        ],
        compiler_params=pltpu.CompilerParams(use_tc_tiling_on_sc=False),
    )
    def k(h_hbm, i_hbm, o_hbm, i_v, r_v, s):
        wid = lax.axis_index("s") * NC + lax.axis_index("c")
        base = wid * B_PER_W
        pltpu.sync_copy(i_hbm.at[pl.ds(base, B_PER_W)], i_v)
        pltpu.async_copy(h_hbm.at[i_v], r_v, s).wait()
        pltpu.sync_copy(r_v, o_hbm.at[pl.ds(base, B_PER_W)])

    return k(H, idx)


def _mlp_body(g1_ref, g2_ref, b1_ref, w2_ref, b2_ref, o_ref):
    h = jnp.maximum(
        g1_ref[:, :HIDDEN] + g2_ref[:, HIDDEN:2 * HIDDEN] + b1_ref[...], 0.0)
    o = jnp.dot(h, w2_ref[...], preferred_element_type=jnp.float32) + b2_ref[...]
    o_ref[...] = jax.nn.sigmoid(o)


def _tc_mlp(g1, g2, b1, W2, b2):
    BLK = 4096
    return pl.pallas_call(
        _mlp_body,
        grid=(BATCH // BLK,),
        in_specs=[
            pl.BlockSpec((BLK, 128), lambda i: (i, 0)),
            pl.BlockSpec((BLK, 128), lambda i: (i, 0)),
            pl.BlockSpec((1, HIDDEN), lambda i: (0, 0)),
            pl.BlockSpec((HIDDEN, 1), lambda i: (0, 0)),
            pl.BlockSpec((1, 1), lambda i: (0, 0)),
        ],
        out_specs=pl.BlockSpec((BLK, 1), lambda i: (i, 0)),
        out_shape=jax.ShapeDtypeStruct((BATCH, 1), jnp.float32),
    )(g1, g2, b1, W2, b2)


def kernel(inputs, E1, E2, W1, b1, W2, b2):
    idx1 = inputs[:, 0]
    idx2 = inputs[:, 1]
    ET1 = E1.T
    ET2 = E2.T
    W1a = W1[:EMB]
    W1b = W1[EMB:]
    H = _embed_matmul(ET1, ET2, W1a, W1b)
    g1 = _sc_gather_one(H, idx1)
    g2 = _sc_gather_one(H, idx2)
    return _tc_mlp(g1, g2, b1.reshape(1, HIDDEN), W2, b2.reshape(1, 1))
